# Initial kernel scaffold; baseline (speedup 1.0000x reference)
import jax
import jax.numpy as jnp
from jax.experimental import pallas as pl


def kernel(x, kernels, hash_a, mode):
    raise NotImplementedError("write your pallas kernel here")


# speedup vs baseline: 5.6803x; 5.6803x over previous
import jax
import jax.numpy as jnp
from jax.experimental import pallas as pl
from jax.experimental.pallas import tpu as pltpu

_IN_C = 96
_OUT_C = 192
_KS = 3
_TBL = 64.0
_NPOW = 5
_D = _KS * _KS * _IN_C
_N = 2
_H = 224
_W = 224
_HT = 8
_MEAN_HT = 32


def _p_row(v):
    n2 = jnp.sum(v * v)
    powers = jnp.stack([n2 ** (2 ** (i + 1)) for i in range(_NPOW)])
    return jnp.concatenate([v, powers])


def _hash_row(a, v):
    proj = jnp.dot(a, v)
    h = jnp.floor(proj)
    return jnp.abs(jnp.fmod(h, _TBL))


def _mean_kernel(x_ref, o_ref):
    n = pl.program_id(0)
    h = pl.program_id(1)

    @pl.when(jnp.logical_and(n == 0, h == 0))
    def _():
        o_ref[...] = jnp.zeros_like(o_ref)

    s = jnp.sum(x_ref[0], axis=(1, 2))
    o_ref[0, :] += s * (1.0 / (_N * _H * _W))


def _mask_kernel(k_ref, a_ref, c_ref, kb_ref, w_ref):
    asum = a_ref[0, 0:_IN_C]
    for j in range(1, _KS * _KS):
        asum = asum + a_ref[0, j * _IN_C:(j + 1) * _IN_C]
    qproj = jnp.sum(c_ref[0, :] * asum)
    qtail = a_ref[0, _D + 0]
    for i in range(1, _NPOW):
        qtail = qtail + a_ref[0, _D + i]
    qproj = qproj + 0.5 * qtail
    hq = jnp.floor(qproj)
    t = hq * (1.0 / _TBL)
    tt = jnp.sign(t) * jnp.floor(jnp.abs(t))
    qb = jnp.abs(hq - tt * _TBL)
    mask = (kb_ref[0, :] == qb).astype(jnp.float32)
    w_ref[...] = k_ref[...] * mask[:, None]


def _conv_kernel(w_ref, x_ref, o_ref, xcol):
    base = pl.program_id(1) * _HT
    for h in range(_HT):
        for kh in range(_KS):
            xr = x_ref[0, :, pl.ds(base + h + kh, 1), :]
            xr2 = xr[:, 0, :]
            for kw in range(_KS):
                kidx = kh * _KS + kw
                xcol[kidx * _IN_C:(kidx + 1) * _IN_C, :] = xr2[:, kw:kw + _W]
        res = jnp.dot(w_ref[...], xcol[...],
                      preferred_element_type=jnp.float32)
        o_ref[0, :, h, :] = res


def kernel(x, kernels, hash_a, mode):
    del mode
    f32 = jnp.float32
    x = x.astype(f32)
    kernels = kernels.astype(f32)
    hash_a = hash_a.astype(f32)

    pk = jax.vmap(lambda kv: _p_row(kv))(kernels)
    kb = jax.vmap(lambda v: _hash_row(hash_a, v))(pk)

    cmean = pl.pallas_call(
        _mean_kernel,
        grid=(_N, _H // _MEAN_HT),
        in_specs=[pl.BlockSpec((1, _IN_C, _MEAN_HT, _W),
                               lambda n, h: (n, 0, h, 0))],
        out_specs=pl.BlockSpec((1, _IN_C), lambda n, h: (0, 0)),
        out_shape=jax.ShapeDtypeStruct((1, _IN_C), f32),
    )(x)

    mw = pl.pallas_call(
        _mask_kernel,
        in_specs=[
            pl.BlockSpec((_OUT_C, _D), lambda: (0, 0)),
            pl.BlockSpec((1, _D + _NPOW), lambda: (0, 0)),
            pl.BlockSpec((1, _IN_C), lambda: (0, 0)),
            pl.BlockSpec((1, _OUT_C), lambda: (0, 0)),
        ],
        out_specs=pl.BlockSpec((_OUT_C, _D), lambda: (0, 0)),
        out_shape=jax.ShapeDtypeStruct((_OUT_C, _D), f32),
    )(kernels, hash_a.reshape(1, _D + _NPOW), cmean, kb.reshape(1, _OUT_C))

    xp = jnp.pad(x, ((0, 0), (0, 0), (1, 1), (1, 1)))
    out = pl.pallas_call(
        _conv_kernel,
        grid=(_N, _H // _HT),
        in_specs=[
            pl.BlockSpec((_OUT_C, _D), lambda n, h: (0, 0)),
            pl.BlockSpec((1, _IN_C, _H + 2, _W + 2), lambda n, h: (n, 0, 0, 0)),
        ],
        out_specs=pl.BlockSpec((1, _OUT_C, _HT, _W), lambda n, h: (n, 0, h, 0)),
        out_shape=jax.ShapeDtypeStruct((_N, _OUT_C, _H, _W), f32),
        scratch_shapes=[pltpu.VMEM((_D, _W), f32)],
    )(mw, xp)
    return out


# speedup vs baseline: 7.6480x; 1.3464x over previous
import jax
import jax.numpy as jnp
from jax.experimental import pallas as pl
from jax.experimental.pallas import tpu as pltpu

_IN_C = 96
_OUT_C = 192
_KS = 3
_TBL = 64.0
_NPOW = 5
_D = _KS * _KS * _IN_C
_CB = _KS * _IN_C
_N = 2
_H = 224
_W = 224
_HT = 8
_MEAN_HT = 32


def _p_row(v):
    n2 = jnp.sum(v * v)
    powers = jnp.stack([n2 ** (2 ** (i + 1)) for i in range(_NPOW)])
    return jnp.concatenate([v, powers])


def _hash_row(a, v):
    proj = jnp.dot(a, v)
    h = jnp.floor(proj)
    return jnp.abs(jnp.fmod(h, _TBL))


def _mean_kernel(x_ref, o_ref, xb_ref):
    n = pl.program_id(0)
    h = pl.program_id(1)

    @pl.when(jnp.logical_and(n == 0, h == 0))
    def _():
        o_ref[...] = jnp.zeros_like(o_ref)

    xv = x_ref[...]
    s = jnp.sum(xv[0], axis=(1, 2))
    o_ref[0, :] += s * (1.0 / (_N * _H * _W))
    xb_ref[0, :, :, 0:_W] = xv[0].astype(jnp.bfloat16)


def _mask_kernel(k_ref, a_ref, c_ref, kb_ref, w_ref):
    asum = a_ref[0, 0:_IN_C]
    for j in range(1, _KS * _KS):
        asum = asum + a_ref[0, j * _IN_C:(j + 1) * _IN_C]
    qproj = jnp.sum(c_ref[0, :] * asum)
    qtail = a_ref[0, _D + 0]
    for i in range(1, _NPOW):
        qtail = qtail + a_ref[0, _D + i]
    qproj = qproj + 0.5 * qtail
    hq = jnp.floor(qproj)
    t = hq * (1.0 / _TBL)
    tt = jnp.sign(t) * jnp.floor(jnp.abs(t))
    qb = jnp.abs(hq - tt * _TBL)
    mask = (kb_ref[0, :] == qb).astype(jnp.float32)
    w_ref[...] = (k_ref[...] * mask[:, None]).astype(jnp.bfloat16)


def _conv_kernel(w_ref, x_ref, o_ref):
    base = pl.program_id(1) * _HT
    lane = jax.lax.broadcasted_iota(jnp.int32, (_IN_C, _W), 1)

    def panel(row, scale=None):
        xr = x_ref[0, :, pl.ds(pl.multiple_of(row * 256, 256), _W)]
        if scale is not None:
            xr = xr * scale
        v0 = jnp.where(lane == 0, jnp.bfloat16(0), jnp.roll(xr, 1, axis=1))
        v2 = jnp.where(lane == _W - 1, jnp.bfloat16(0),
                       jnp.roll(xr, -1, axis=1))
        return jnp.concatenate([v0, xr, v2], axis=0)

    vp = (base > 0).astype(jnp.bfloat16)
    vn = (base + _HT < _H).astype(jnp.bfloat16)
    panels = [panel(jnp.maximum(base - 1, 0), vp)]
    for j in range(_HT):
        panels.append(panel(base + j))
    panels.append(panel(jnp.minimum(base + _HT, _H - 1), vn))

    wv = w_ref[...]
    for h in range(_HT):
        rhs = jnp.concatenate([panels[h], panels[h + 1], panels[h + 2]],
                              axis=0)
        o_ref[0, :, h, :] = jnp.dot(wv, rhs,
                                    preferred_element_type=jnp.float32)


def kernel(x, kernels, hash_a, mode):
    del mode
    f32 = jnp.float32
    x = x.astype(f32)
    kernels = kernels.astype(f32)
    hash_a = hash_a.astype(f32)

    pk = jax.vmap(lambda kv: _p_row(kv))(kernels)
    kb = jax.vmap(lambda v: _hash_row(hash_a, v))(pk)

    cmean, xb = pl.pallas_call(
        _mean_kernel,
        grid=(_N, _H // _MEAN_HT),
        in_specs=[pl.BlockSpec((1, _IN_C, _MEAN_HT, _W),
                               lambda n, h: (n, 0, h, 0))],
        out_specs=[
            pl.BlockSpec((1, _IN_C), lambda n, h: (0, 0)),
            pl.BlockSpec((1, _IN_C, _MEAN_HT, 256), lambda n, h: (n, 0, h, 0)),
        ],
        out_shape=[
            jax.ShapeDtypeStruct((1, _IN_C), f32),
            jax.ShapeDtypeStruct((_N, _IN_C, _H, 256), jnp.bfloat16),
        ],
    )(x)

    w3 = pl.pallas_call(
        _mask_kernel,
        in_specs=[
            pl.BlockSpec((_OUT_C, _D), lambda: (0, 0)),
            pl.BlockSpec((1, _D + _NPOW), lambda: (0, 0)),
            pl.BlockSpec((1, _IN_C), lambda: (0, 0)),
            pl.BlockSpec((1, _OUT_C), lambda: (0, 0)),
        ],
        out_specs=pl.BlockSpec((_OUT_C, _D), lambda: (0, 0)),
        out_shape=jax.ShapeDtypeStruct((_OUT_C, _D), jnp.bfloat16),
    )(kernels, hash_a.reshape(1, _D + _NPOW), cmean, kb.reshape(1, _OUT_C))

    xb2 = xb.reshape(_N, _IN_C, _H * 256)
    out = pl.pallas_call(
        _conv_kernel,
        grid=(_N, _H // _HT),
        in_specs=[
            pl.BlockSpec((_OUT_C, _D), lambda n, hi: (0, 0)),
            pl.BlockSpec((1, _IN_C, _H * 256), lambda n, hi: (n, 0, 0)),
        ],
        out_specs=pl.BlockSpec((1, _OUT_C, _HT, _W),
                               lambda n, hi: (n, 0, hi, 0)),
        out_shape=jax.ShapeDtypeStruct((_N, _OUT_C, _H, _W), f32),
    )(w3, xb2)
    return out
